# baseline (speedup 1.0000x reference)
import jax
import jax.numpy as jnp
from jax import lax
from jax.experimental import pallas as pl
from jax.experimental.pallas import tpu as pltpu
from jax.experimental.pallas import tpu_sc as plsc

_V = 1000000
_D = 64
_SCALE = float(_D) ** 0.5
_L = 16

_NW = 32
_SEQ = 16384
_SLEN = 50
_SEQ_PER_W = _SEQ // _NW

_NBUF = 4
_LA = 2


def _gather_body(x_hbm, table_hbm, out_hbm, idx_v, buf_v, gsems, ssems):
    wid = lax.axis_index("s") * 2 + lax.axis_index("c")
    seq0 = wid * _SEQ_PER_W

    pltpu.sync_copy(x_hbm.at[pl.ds(seq0, _SEQ_PER_W)], idx_v)

    def start_gather(j, b):
        pltpu.make_async_copy(
            table_hbm.at[idx_v.at[j]], buf_v.at[b], gsems.at[b]
        ).start()

    def wait_gather(b):
        pltpu.make_async_copy(
            table_hbm.at[idx_v.at[0]], buf_v.at[b], gsems.at[b]
        ).wait()

    def start_store(j, b):
        pltpu.make_async_copy(
            buf_v.at[b], out_hbm.at[seq0 + j], ssems.at[b]
        ).start()

    def wait_store(b):
        pltpu.make_async_copy(
            buf_v.at[b], out_hbm.at[seq0], ssems.at[b]
        ).wait()

    def scale(b):
        def body(i, _):
            r = i * 2
            for k in range(2):
                for c in range(_D // _L):
                    sl = pl.ds(c * _L, _L)
                    buf_v[b, r + k, sl] = buf_v[b, r + k, sl] * _SCALE
            return 0

        lax.fori_loop(0, _SLEN // 2, body, 0)


    def emit(j, b, g, need_store_wait):
        wait_gather(b)
        scale(b)
        start_store(j, b)
        if g is not None:
            b2 = (b + _LA) % _NBUF
            if need_store_wait:
                wait_store(b2)
            start_gather(g, b2)

    for g in range(_LA):
        start_gather(g, g % _NBUF)

    for j in range(_NBUF):
        emit(j, j % _NBUF, j + _LA, j + _LA >= _NBUF)

    n_groups = (_SEQ_PER_W - _NBUF - _LA) // _NBUF

    def steady(t, _):
        j0 = _NBUF + t * _NBUF
        for i in range(_NBUF):
            emit(j0 + i, i, j0 + i + _LA, True)
        return 0

    lax.fori_loop(0, n_groups, steady, 0)

    for j in range(_NBUF + n_groups * _NBUF, _SEQ_PER_W):
        g = j + _LA
        emit(j, j % _NBUF, g if g < _SEQ_PER_W else None, True)

    for b in range(_NBUF):
        wait_store(b)


@jax.jit
def _embed(x, embed_weight):
    mesh = plsc.VectorSubcoreMesh(core_axis_name="c", subcore_axis_name="s")
    run = pl.kernel(
        _gather_body,
        out_type=jax.ShapeDtypeStruct((_SEQ, _SLEN, _D), jnp.float32),
        mesh=mesh,
        scratch_types=[
            pltpu.VMEM((_SEQ_PER_W, _SLEN), jnp.int32),
            pltpu.VMEM((_NBUF, _SLEN, _D), jnp.float32),
            pltpu.SemaphoreType.DMA((_NBUF,)),
            pltpu.SemaphoreType.DMA((_NBUF,)),
        ],
        compiler_params=pltpu.CompilerParams(use_tc_tiling_on_sc=False),
    )
    return run(x, embed_weight)


def kernel(x, embed_weight):
    return _embed(x.astype(jnp.int32), embed_weight)
